# baseline (speedup 1.0000x reference)
import jax
import jax.numpy as jnp
from jax import lax
from jax.experimental import pallas as pl
from jax.experimental.pallas import tpu as pltpu
from jax.experimental.pallas import tpu_sc as plsc

_NT = 26
_NWT = 2
_NTT = _NT + _NWT
_NTA = 13
_NTB = _NT - _NTA
_VOCAB = 100000
_VS = 100096
_DIM = 16
_B = 4096
_L = 20
_NF = 10
_NC = 2
_NS = 16
_NW = _NC * _NS
_SAMP = _B // _NW
_CH = 128
_PD = _NTT * _DIM

_VC = 5888
_RPB = _VC * _DIM // 128
_RPT = _VS * _DIM // 128


def _fmt_body(src, dst):
    x = src[0]
    z = jnp.concatenate([x[:, i * _RPB:(i + 1) * _RPB] for i in range(8)],
                        axis=0)
    dst[...] = jnp.transpose(z, (1, 0))


def _fmt(tblT, nt):
    return pl.pallas_call(
        _fmt_body,
        grid=(nt, _VS // _VC),
        in_specs=[pl.BlockSpec((1, _DIM, _VC), lambda t, v: (t, 0, v))],
        out_specs=pl.BlockSpec((_RPB, 128), lambda t, v: (t * (_VS // _VC) + v, 0)),
        out_shape=jax.ShapeDtypeStruct((nt * _RPT, 128), jnp.float32),
    )(tblT)


def _pool_tables(idxT, tbl, out, idx_v, pool_v, sem, nt, b0, col0):
    def table_body(t, carry):
        pltpu.sync_copy(idxT.at[t, :, pl.ds(b0, _SAMP)], idx_v)
        pltpu.async_copy(tbl.at[idx_v.at[0]], pool_v, sem).wait()
        hs = [pltpu.async_copy(tbl.at[idx_v.at[j]], pool_v, sem, add=True)
              for j in range(1, _L)]
        for h in hs:
            h.wait()
        toff = pl.multiple_of(col0 + t * _DIM, _DIM)
        pltpu.sync_copy(pool_v, out.at[pl.ds(b0, _SAMP), pl.ds(toff, _DIM)])
        return carry

    lax.fori_loop(0, nt, table_body, 0)


def _sc_pool_a(idlT, emb, out, idx_v, pool_v, sem):
    wid = lax.axis_index("s") * _NC + lax.axis_index("c")
    b0 = wid * _SAMP
    _pool_tables(idlT, emb, out, idx_v, pool_v, sem, _NTA, b0, 0)


def _sc_pool_b(idlT, idsT, wts, emb, wemb, out, idx_v, rows_v, wts_v, pool_v,
               sem):
    wid = lax.axis_index("s") * _NC + lax.axis_index("c")
    b0 = wid * _SAMP
    _pool_tables(idlT, emb, out, idx_v, pool_v, sem, _NTB, b0, 0)

    dn = lax.GatherDimensionNumbers(offset_dims=(), collapsed_slice_dims=(0,),
                                    start_index_map=(0,))
    for w in range(_NWT):
        pltpu.sync_copy(idsT.at[w, :, pl.ds(b0, _SAMP)], idx_v)
        pltpu.sync_copy(wts.at[w, pl.ds(b0 * _L, _SAMP * _L)], wts_v)
        hs = [pltpu.async_copy(wemb.at[idx_v.at[j]],
                               rows_v.at[pl.ds(pl.multiple_of(j * _CH, _CH), _CH), :],
                               sem)
              for j in range(_L)]
        for h in hs:
            h.wait()

        def wbag(s, c):
            acc = None
            for j in range(_L):
                r = s * _L + j
                q = r // 16
                lane = jnp.full((16, 1), r - q * 16, jnp.int32)
                wvec = wts_v[pl.ds(pl.multiple_of(q * 16, 16), 16)]
                wb = lax.gather(wvec, lane, dn, (1,),
                                mode=lax.GatherScatterMode.PROMISE_IN_BOUNDS)
                rv = rows_v[j * _CH + s, :] * wb
                acc = rv if acc is None else acc + rv
            pool_v[s, :] = acc
            return c

        lax.fori_loop(0, _SAMP, wbag, 0)
        pltpu.sync_copy(pool_v,
                        out.at[pl.ds(b0, _SAMP),
                               pl.ds((_NTB + w) * _DIM, _DIM)])


_mesh = plsc.VectorSubcoreMesh(core_axis_name="c", subcore_axis_name="s")
_sc_params = pltpu.CompilerParams(use_tc_tiling_on_sc=False)

_sc_pool_a_call = pl.kernel(
    _sc_pool_a,
    out_type=jax.ShapeDtypeStruct((_B, _NTA * _DIM), jnp.float32),
    mesh=_mesh,
    compiler_params=_sc_params,
    scratch_types=[
        pltpu.VMEM((_L, _CH), jnp.int32),
        pltpu.VMEM((_SAMP, _DIM), jnp.float32),
        pltpu.SemaphoreType.DMA,
    ],
)

_sc_pool_b_call = pl.kernel(
    _sc_pool_b,
    out_type=jax.ShapeDtypeStruct((_B, (_NTB + _NWT) * _DIM), jnp.float32),
    mesh=_mesh,
    compiler_params=_sc_params,
    scratch_types=[
        pltpu.VMEM((_L, _CH), jnp.int32),
        pltpu.VMEM((_SAMP * _L, _DIM), jnp.float32),
        pltpu.VMEM((_SAMP * _L,), jnp.float32),
        pltpu.VMEM((_SAMP, _DIM), jnp.float32),
        pltpu.SemaphoreType.DMA,
    ],
)


_PDA = _NTA * _DIM
_PDB = (_NTB + _NWT) * _DIM


def _tc_body(ff, pa, pb, wd, bd, wo, bo, out):
    dense = jnp.dot(ff[...], wd[...], preferred_element_type=jnp.float32) + bd[...]
    r = jnp.dot(dense, wo[0:8, :], preferred_element_type=jnp.float32)
    r = r + jnp.dot(pa[...], wo[8:8 + _PDA, :], preferred_element_type=jnp.float32)
    r = r + jnp.dot(pb[...], wo[8 + _PDA:, :], preferred_element_type=jnp.float32)
    out[...] = r + bo[...]


_BLK = 512


def _tc_mix(ff, pa, pb, wd, bd, wo, bo):
    return pl.pallas_call(
        _tc_body,
        grid=(_B // _BLK,),
        in_specs=[
            pl.BlockSpec((_BLK, _NF), lambda i: (i, 0)),
            pl.BlockSpec((_BLK, _PDA), lambda i: (i, 0)),
            pl.BlockSpec((_BLK, _PDB), lambda i: (i, 0)),
            pl.BlockSpec((_NF, 8), lambda i: (0, 0)),
            pl.BlockSpec((1, 8), lambda i: (0, 0)),
            pl.BlockSpec((8 + _PD, _DIM), lambda i: (0, 0)),
            pl.BlockSpec((1, _DIM), lambda i: (0, 0)),
        ],
        out_specs=pl.BlockSpec((_BLK, _DIM), lambda i: (i, 0)),
        out_shape=jax.ShapeDtypeStruct((_B, _DIM), jnp.float32),
    )(ff, pa, pb, wd, bd, wo, bo)


def _remap(idxT, nt):
    tb = (jnp.arange(nt, dtype=jnp.int32) * (_VS // _VC))[:, None, None]
    vb = idxT // _VC
    vr = idxT - vb * _VC
    return ((tb + vb) * _RPB + vr % _RPB) * 8 + vr // _RPB


def kernel(float_features, idlist_indices, idscore_indices, idscore_weights,
           emb_tables, w_emb_tables, W_dense, b_dense, W_over, b_over):
    idlT = jnp.transpose(idlist_indices, (0, 2, 1))
    idlT_a = _remap(idlT[:_NTA], _NTA)
    idlT_b = _remap(idlT[_NTA:], _NTB)
    idsT = _remap(jnp.transpose(idscore_indices, (0, 2, 1)), _NWT)
    wts = idscore_weights.reshape(_NWT, _B * _L)
    embT = jnp.transpose(emb_tables, (0, 2, 1))
    wemb = _fmt(jnp.transpose(w_emb_tables, (0, 2, 1)), _NWT
                ).reshape(_NWT * _VS, _DIM)
    emb_a = _fmt(embT[:_NTA], _NTA).reshape(_NTA * _VS, _DIM)
    emb_b = _fmt(embT[_NTA:], _NTB).reshape(_NTB * _VS, _DIM)
    pa = _sc_pool_a_call(idlT_a, emb_a)
    pb = _sc_pool_b_call(idlT_b, idsT, wts, emb_b, wemb)
    return _tc_mix(float_features, pa, pb, W_dense, b_dense.reshape(1, 8),
                   W_over, b_over.reshape(1, 16))


# speedup vs baseline: 1.1973x; 1.1973x over previous
import jax
import jax.numpy as jnp
from jax import lax
from jax.experimental import pallas as pl
from jax.experimental.pallas import tpu as pltpu
from jax.experimental.pallas import tpu_sc as plsc

_NT = 26
_NWT = 2
_NTT = _NT + _NWT
_NTA = 16
_NTB = _NT - _NTA
_VOCAB = 100000
_VS = 100096
_DIM = 16
_B = 4096
_L = 20
_NF = 10
_NC = 2
_NS = 16
_NW = _NC * _NS
_SAMP = _B // _NW
_CH = 128
_PD = _NTT * _DIM

_VC = 5888
_RPB = _VC * _DIM // 128
_RPT = _VS * _DIM // 128


def _fmt_body(src, dst):
    x = src[0]
    z = jnp.concatenate([x[:, i * _RPB:(i + 1) * _RPB] for i in range(8)],
                        axis=0)
    dst[...] = jnp.transpose(z, (1, 0))


def _fmt(tblT, nt, t0=0):
    return pl.pallas_call(
        _fmt_body,
        grid=(nt, _VS // _VC),
        in_specs=[pl.BlockSpec((1, _DIM, _VC), lambda t, v: (t + t0, 0, v))],
        out_specs=pl.BlockSpec((_RPB, 128), lambda t, v: (t * (_VS // _VC) + v, 0)),
        out_shape=jax.ShapeDtypeStruct((nt * _RPT, 128), jnp.float32),
    )(tblT)


def _pool_tables(idxT, tbl, out, idx_v, pool_v, sem, nt, b0, col0):
    def table_body(t, carry):
        pltpu.sync_copy(idxT.at[t, :, pl.ds(b0, _SAMP)], idx_v)
        pltpu.async_copy(tbl.at[idx_v.at[0]], pool_v, sem).wait()
        hs = [pltpu.async_copy(tbl.at[idx_v.at[j]], pool_v, sem, add=True)
              for j in range(1, _L)]
        for h in hs:
            h.wait()
        toff = pl.multiple_of(col0 + t * _DIM, _DIM)
        pltpu.sync_copy(pool_v, out.at[pl.ds(b0, _SAMP), pl.ds(toff, _DIM)])
        return carry

    lax.fori_loop(0, nt, table_body, 0)


def _sc_pool_a(idlT, emb, out, idx_v, pool_v, sem):
    wid = lax.axis_index("s") * _NC + lax.axis_index("c")
    b0 = wid * _SAMP
    _pool_tables(idlT, emb, out, idx_v, pool_v, sem, _NTA, b0, 0)


def _sc_pool_b(idlT, idsT, wts, emb, wemb, out, idx_v, rows_v, wts_v, pool_v,
               sem):
    wid = lax.axis_index("s") * _NC + lax.axis_index("c")
    b0 = wid * _SAMP
    _pool_tables(idlT, emb, out, idx_v, pool_v, sem, _NTB, b0, 0)

    dn = lax.GatherDimensionNumbers(offset_dims=(), collapsed_slice_dims=(0,),
                                    start_index_map=(0,))
    for w in range(_NWT):
        pltpu.sync_copy(idsT.at[w, :, pl.ds(b0, _SAMP)], idx_v)
        pltpu.sync_copy(wts.at[w, pl.ds(b0 * _L, _SAMP * _L)], wts_v)
        hs = [pltpu.async_copy(wemb.at[idx_v.at[j]],
                               rows_v.at[pl.ds(pl.multiple_of(j * _CH, _CH), _CH), :],
                               sem)
              for j in range(_L)]
        for h in hs:
            h.wait()

        def wbag(s, c):
            acc = None
            for j in range(_L):
                r = s * _L + j
                q = r // 16
                lane = jnp.full((16, 1), r - q * 16, jnp.int32)
                wvec = wts_v[pl.ds(pl.multiple_of(q * 16, 16), 16)]
                wb = lax.gather(wvec, lane, dn, (1,),
                                mode=lax.GatherScatterMode.PROMISE_IN_BOUNDS)
                rv = rows_v[j * _CH + s, :] * wb
                acc = rv if acc is None else acc + rv
            pool_v[s, :] = acc
            return c

        lax.fori_loop(0, _SAMP, wbag, 0)
        pltpu.sync_copy(pool_v,
                        out.at[pl.ds(b0, _SAMP),
                               pl.ds((_NTB + w) * _DIM, _DIM)])


_mesh = plsc.VectorSubcoreMesh(core_axis_name="c", subcore_axis_name="s")
_sc_params = pltpu.CompilerParams(use_tc_tiling_on_sc=False)

_sc_pool_a_call = pl.kernel(
    _sc_pool_a,
    out_type=jax.ShapeDtypeStruct((_B, _NTA * _DIM), jnp.float32),
    mesh=_mesh,
    compiler_params=_sc_params,
    scratch_types=[
        pltpu.VMEM((_L, _CH), jnp.int32),
        pltpu.VMEM((_SAMP, _DIM), jnp.float32),
        pltpu.SemaphoreType.DMA,
    ],
)

_sc_pool_b_call = pl.kernel(
    _sc_pool_b,
    out_type=jax.ShapeDtypeStruct((_B, (_NTB + _NWT) * _DIM), jnp.float32),
    mesh=_mesh,
    compiler_params=_sc_params,
    scratch_types=[
        pltpu.VMEM((_L, _CH), jnp.int32),
        pltpu.VMEM((_SAMP * _L, _DIM), jnp.float32),
        pltpu.VMEM((_SAMP * _L,), jnp.float32),
        pltpu.VMEM((_SAMP, _DIM), jnp.float32),
        pltpu.SemaphoreType.DMA,
    ],
)


_PDA = _NTA * _DIM
_PDB = (_NTB + _NWT) * _DIM


def _tc_body(ff, pa, pb, wd, bd, wo, bo, out):
    dense = jnp.dot(ff[...], wd[...], preferred_element_type=jnp.float32) + bd[...]
    r = jnp.dot(dense, wo[0:8, :], preferred_element_type=jnp.float32)
    r = r + jnp.dot(pa[...], wo[8:8 + _PDA, :], preferred_element_type=jnp.float32)
    r = r + jnp.dot(pb[...], wo[8 + _PDA:, :], preferred_element_type=jnp.float32)
    out[...] = r + bo[...]


_BLK = 512


def _tc_mix(ff, pa, pb, wd, bd, wo, bo):
    return pl.pallas_call(
        _tc_body,
        grid=(_B // _BLK,),
        in_specs=[
            pl.BlockSpec((_BLK, _NF), lambda i: (i, 0)),
            pl.BlockSpec((_BLK, _PDA), lambda i: (i, 0)),
            pl.BlockSpec((_BLK, _PDB), lambda i: (i, 0)),
            pl.BlockSpec((_NF, 8), lambda i: (0, 0)),
            pl.BlockSpec((1, 8), lambda i: (0, 0)),
            pl.BlockSpec((8 + _PD, _DIM), lambda i: (0, 0)),
            pl.BlockSpec((1, _DIM), lambda i: (0, 0)),
        ],
        out_specs=pl.BlockSpec((_BLK, _DIM), lambda i: (i, 0)),
        out_shape=jax.ShapeDtypeStruct((_B, _DIM), jnp.float32),
    )(ff, pa, pb, wd, bd, wo, bo)


def _remap(idxT, nt):
    tb = (jnp.arange(nt, dtype=jnp.int32) * (_VS // _VC))[:, None, None]
    vb = idxT // _VC
    vr = idxT - vb * _VC
    return ((tb + vb) * _RPB + vr % _RPB) * 8 + vr // _RPB


def kernel(float_features, idlist_indices, idscore_indices, idscore_weights,
           emb_tables, w_emb_tables, W_dense, b_dense, W_over, b_over):
    idlT = jnp.transpose(idlist_indices, (0, 2, 1))
    idlT_a = _remap(idlT[:_NTA], _NTA)
    idlT_b = _remap(idlT[_NTA:], _NTB)
    idsT = _remap(jnp.transpose(idscore_indices, (0, 2, 1)), _NWT)
    wts = idscore_weights.reshape(_NWT, _B * _L)
    embT = jnp.transpose(emb_tables, (0, 2, 1))
    wemb = _fmt(jnp.transpose(w_emb_tables, (0, 2, 1)), _NWT
                ).reshape(_NWT * _VS, _DIM)
    emb_a = _fmt(embT, _NTA, 0).reshape(_NTA * _VS, _DIM)
    emb_b = _fmt(embT, _NTB, _NTA).reshape(_NTB * _VS, _DIM)
    pa = _sc_pool_a_call(idlT_a, emb_a)
    pb = _sc_pool_b_call(idlT_b, idsT, wts, emb_b, wemb)
    return _tc_mix(float_features, pa, pb, W_dense, b_dense.reshape(1, 8),
                   W_over, b_over.reshape(1, 16))


# speedup vs baseline: 1.2367x; 1.0329x over previous
import jax
import jax.numpy as jnp
from jax import lax
from jax.experimental import pallas as pl
from jax.experimental.pallas import tpu as pltpu
from jax.experimental.pallas import tpu_sc as plsc

_NT = 26
_NWT = 2
_NTT = _NT + _NWT
_NTA = 16
_NTB = _NT - _NTA
_VOCAB = 100000
_VS = 100096
_DIM = 16
_B = 4096
_L = 20
_NF = 10
_NC = 2
_NS = 16
_NW = _NC * _NS
_SAMP = _B // _NW
_CH = 128
_PD = _NTT * _DIM

_VC = 5888
_RPB = _VC * _DIM // 128
_RPT = _VS * _DIM // 128


def _fmt_body(src, dst):
    x = src[0]
    z = jnp.concatenate([x[:, i * _RPB:(i + 1) * _RPB] for i in range(8)],
                        axis=0)
    dst[...] = jnp.transpose(z, (1, 0))


def _fmt(tblT, nt, t0=0):
    return pl.pallas_call(
        _fmt_body,
        grid=(nt, _VS // _VC),
        in_specs=[pl.BlockSpec((1, _DIM, _VC), lambda t, v: (t + t0, 0, v))],
        out_specs=pl.BlockSpec((_RPB, 128), lambda t, v: (t * (_VS // _VC) + v, 0)),
        out_shape=jax.ShapeDtypeStruct((nt * _RPT, 128), jnp.float32),
    )(tblT)


def _pool_tables(idxT, tbl, out, idx_v, pool_v, sem, nt, b0, col0):
    def table_body(t, carry):
        pltpu.sync_copy(idxT.at[t, :, pl.ds(b0, _SAMP)], idx_v)
        pltpu.async_copy(tbl.at[idx_v.at[0]], pool_v, sem).wait()
        hs = [pltpu.async_copy(tbl.at[idx_v.at[j]], pool_v, sem, add=True)
              for j in range(1, _L)]
        for h in hs:
            h.wait()
        toff = pl.multiple_of(col0 + t * _DIM, _DIM)
        pltpu.sync_copy(pool_v, out.at[pl.ds(b0, _SAMP), pl.ds(toff, _DIM)])
        return carry

    lax.fori_loop(0, nt, table_body, 0)


def _sc_pool_b(idlT, emb, out, idx_v, pool_v, sem):
    wid = lax.axis_index("s") * _NC + lax.axis_index("c")
    b0 = wid * _SAMP
    _pool_tables(idlT, emb, out, idx_v, pool_v, sem, _NTB, b0, 0)


def _sc_pool_a(idlT, idsT, wts, emb, wemb, out, idx_v, rows_v, wts_v, pool_v,
               sem):
    wid = lax.axis_index("s") * _NC + lax.axis_index("c")
    b0 = wid * _SAMP
    _pool_tables(idlT, emb, out, idx_v, pool_v, sem, _NTA, b0, 0)

    dn = lax.GatherDimensionNumbers(offset_dims=(), collapsed_slice_dims=(0,),
                                    start_index_map=(0,))
    for w in range(_NWT):
        pltpu.sync_copy(idsT.at[w, :, pl.ds(b0, _SAMP)], idx_v)
        pltpu.sync_copy(wts.at[w, pl.ds(b0 * _L, _SAMP * _L)], wts_v)
        hs = [pltpu.async_copy(wemb.at[idx_v.at[j]],
                               rows_v.at[pl.ds(pl.multiple_of(j * _CH, _CH), _CH), :],
                               sem)
              for j in range(_L)]
        for h in hs:
            h.wait()

        def wbag(s, c):
            acc = None
            for j in range(_L):
                r = s * _L + j
                q = r // 16
                lane = jnp.full((16, 1), r - q * 16, jnp.int32)
                wvec = wts_v[pl.ds(pl.multiple_of(q * 16, 16), 16)]
                wb = lax.gather(wvec, lane, dn, (1,),
                                mode=lax.GatherScatterMode.PROMISE_IN_BOUNDS)
                rv = rows_v[j * _CH + s, :] * wb
                acc = rv if acc is None else acc + rv
            pool_v[s, :] = acc
            return c

        lax.fori_loop(0, _SAMP, wbag, 0)
        pltpu.sync_copy(pool_v,
                        out.at[pl.ds(b0, _SAMP),
                               pl.ds((_NTA + w) * _DIM, _DIM)])


_mesh = plsc.VectorSubcoreMesh(core_axis_name="c", subcore_axis_name="s")
_sc_params = pltpu.CompilerParams(use_tc_tiling_on_sc=False)

_sc_pool_b_call = pl.kernel(
    _sc_pool_b,
    out_type=jax.ShapeDtypeStruct((_B, _NTB * _DIM), jnp.float32),
    mesh=_mesh,
    compiler_params=_sc_params,
    scratch_types=[
        pltpu.VMEM((_L, _CH), jnp.int32),
        pltpu.VMEM((_SAMP, _DIM), jnp.float32),
        pltpu.SemaphoreType.DMA,
    ],
)

_sc_pool_a_call = pl.kernel(
    _sc_pool_a,
    out_type=jax.ShapeDtypeStruct((_B, (_NTA + _NWT) * _DIM), jnp.float32),
    mesh=_mesh,
    compiler_params=_sc_params,
    scratch_types=[
        pltpu.VMEM((_L, _CH), jnp.int32),
        pltpu.VMEM((_SAMP * _L, _DIM), jnp.float32),
        pltpu.VMEM((_SAMP * _L,), jnp.float32),
        pltpu.VMEM((_SAMP, _DIM), jnp.float32),
        pltpu.SemaphoreType.DMA,
    ],
)


_PDA = (_NTA + _NWT) * _DIM
_PDB = _NTB * _DIM


def _tc_body(ff, pa, pb, wd, bd, wo, bo, out):
    dense = jnp.dot(ff[...], wd[...], preferred_element_type=jnp.float32) + bd[...]
    r = jnp.dot(dense, wo[0:8, :], preferred_element_type=jnp.float32)
    r = r + jnp.dot(pa[:, :_NTA * _DIM], wo[8:8 + _NTA * _DIM, :],
                    preferred_element_type=jnp.float32)
    r = r + jnp.dot(pa[:, _NTA * _DIM:], wo[8 + _NT * _DIM:, :],
                    preferred_element_type=jnp.float32)
    r = r + jnp.dot(pb[...], wo[8 + _NTA * _DIM:8 + _NT * _DIM, :],
                    preferred_element_type=jnp.float32)
    out[...] = r + bo[...]


_BLK = 512


def _tc_mix(ff, pa, pb, wd, bd, wo, bo):
    return pl.pallas_call(
        _tc_body,
        grid=(_B // _BLK,),
        in_specs=[
            pl.BlockSpec((_BLK, _NF), lambda i: (i, 0)),
            pl.BlockSpec((_BLK, _PDA), lambda i: (i, 0)),
            pl.BlockSpec((_BLK, _PDB), lambda i: (i, 0)),
            pl.BlockSpec((_NF, 8), lambda i: (0, 0)),
            pl.BlockSpec((1, 8), lambda i: (0, 0)),
            pl.BlockSpec((8 + _PD, _DIM), lambda i: (0, 0)),
            pl.BlockSpec((1, _DIM), lambda i: (0, 0)),
        ],
        out_specs=pl.BlockSpec((_BLK, _DIM), lambda i: (i, 0)),
        out_shape=jax.ShapeDtypeStruct((_B, _DIM), jnp.float32),
    )(ff, pa, pb, wd, bd, wo, bo)


def _remap(idxT, nt):
    tb = (jnp.arange(nt, dtype=jnp.int32) * (_VS // _VC))[:, None, None]
    vb = idxT // _VC
    vr = idxT - vb * _VC
    return ((tb + vb) * _RPB + vr % _RPB) * 8 + vr // _RPB


def kernel(float_features, idlist_indices, idscore_indices, idscore_weights,
           emb_tables, w_emb_tables, W_dense, b_dense, W_over, b_over):
    idlT = jnp.transpose(idlist_indices, (0, 2, 1))
    idlT_a = _remap(idlT[:_NTA], _NTA)
    idlT_b = _remap(idlT[_NTA:], _NTB)
    idsT = _remap(jnp.transpose(idscore_indices, (0, 2, 1)), _NWT)
    wts = idscore_weights.reshape(_NWT, _B * _L)
    embT = jnp.transpose(emb_tables, (0, 2, 1))
    wemb = _fmt(jnp.transpose(w_emb_tables, (0, 2, 1)), _NWT
                ).reshape(_NWT * _VS, _DIM)
    emb_a = _fmt(embT, _NTA, 0).reshape(_NTA * _VS, _DIM)
    emb_b = _fmt(embT, _NTB, _NTA).reshape(_NTB * _VS, _DIM)
    pa = _sc_pool_a_call(idlT_a, idsT, wts, emb_a, wemb)
    pb = _sc_pool_b_call(idlT_b, emb_b)
    return _tc_mix(float_features, pa, pb, W_dense, b_dense.reshape(1, 8),
                   W_over, b_over.reshape(1, 16))


# speedup vs baseline: 1.2733x; 1.0296x over previous
import jax
import jax.numpy as jnp
from jax import lax
from jax.experimental import pallas as pl
from jax.experimental.pallas import tpu as pltpu
from jax.experimental.pallas import tpu_sc as plsc

_NT = 26
_NWT = 2
_NTT = _NT + _NWT
_NTA = 10
_NTB = 8
_NTC = _NT - _NTA - _NTB
_VOCAB = 100000
_VS = 100096
_DIM = 16
_B = 4096
_L = 20
_NF = 10
_NC = 2
_NS = 16
_NW = _NC * _NS
_SAMP = _B // _NW
_CH = 128
_PD = _NTT * _DIM

_VC = 5888
_RPB = _VC * _DIM // 128
_RPT = _VS * _DIM // 128


def _fmt_body(src, dst):
    x = src[0]
    z = jnp.concatenate([x[:, i * _RPB:(i + 1) * _RPB] for i in range(8)],
                        axis=0)
    dst[...] = jnp.transpose(z, (1, 0))


def _fmt(tblT, nt, t0=0):
    return pl.pallas_call(
        _fmt_body,
        grid=(nt, _VS // _VC),
        in_specs=[pl.BlockSpec((1, _DIM, _VC), lambda t, v: (t + t0, 0, v))],
        out_specs=pl.BlockSpec((_RPB, 128), lambda t, v: (t * (_VS // _VC) + v, 0)),
        out_shape=jax.ShapeDtypeStruct((nt * _RPT, 128), jnp.float32),
    )(tblT)


def _pool_tables(idxT, tbl, out, idx_v, pool_v, sem, nt, b0, col0):
    def table_body(t, carry):
        pltpu.sync_copy(idxT.at[t, :, pl.ds(b0, _SAMP)], idx_v)
        pltpu.async_copy(tbl.at[idx_v.at[0]], pool_v, sem).wait()
        hs = [pltpu.async_copy(tbl.at[idx_v.at[j]], pool_v, sem, add=True)
              for j in range(1, _L)]
        for h in hs:
            h.wait()
        toff = pl.multiple_of(col0 + t * _DIM, _DIM)
        pltpu.sync_copy(pool_v, out.at[pl.ds(b0, _SAMP), pl.ds(toff, _DIM)])
        return carry

    lax.fori_loop(0, nt, table_body, 0)


def _make_plain_pool(nt):
    def _sc_pool_plain(idlT, emb, out, idx_v, pool_v, sem):
        wid = lax.axis_index("s") * _NC + lax.axis_index("c")
        b0 = wid * _SAMP
        _pool_tables(idlT, emb, out, idx_v, pool_v, sem, nt, b0, 0)

    return pl.kernel(
        _sc_pool_plain,
        out_type=jax.ShapeDtypeStruct((_B, nt * _DIM), jnp.float32),
        mesh=_mesh,
        compiler_params=_sc_params,
        scratch_types=[
            pltpu.VMEM((_L, _CH), jnp.int32),
            pltpu.VMEM((_SAMP, _DIM), jnp.float32),
            pltpu.SemaphoreType.DMA,
        ],
    )


def _sc_pool_a(idlT, idsT, wts, emb, wemb, out, idx_v, rows_v, wts_v, pool_v,
               sem):
    wid = lax.axis_index("s") * _NC + lax.axis_index("c")
    b0 = wid * _SAMP
    _pool_tables(idlT, emb, out, idx_v, pool_v, sem, _NTA, b0, 0)

    dn = lax.GatherDimensionNumbers(offset_dims=(), collapsed_slice_dims=(0,),
                                    start_index_map=(0,))
    for w in range(_NWT):
        pltpu.sync_copy(idsT.at[w, :, pl.ds(b0, _SAMP)], idx_v)
        pltpu.sync_copy(wts.at[w, pl.ds(b0 * _L, _SAMP * _L)], wts_v)
        hs = [pltpu.async_copy(wemb.at[idx_v.at[j]],
                               rows_v.at[pl.ds(pl.multiple_of(j * _CH, _CH), _CH), :],
                               sem)
              for j in range(_L)]
        for h in hs:
            h.wait()

        def wbag(s, c):
            acc = None
            for j in range(_L):
                r = s * _L + j
                q = r // 16
                lane = jnp.full((16, 1), r - q * 16, jnp.int32)
                wvec = wts_v[pl.ds(pl.multiple_of(q * 16, 16), 16)]
                wb = lax.gather(wvec, lane, dn, (1,),
                                mode=lax.GatherScatterMode.PROMISE_IN_BOUNDS)
                rv = rows_v[j * _CH + s, :] * wb
                acc = rv if acc is None else acc + rv
            pool_v[s, :] = acc
            return c

        lax.fori_loop(0, _SAMP, wbag, 0)
        pltpu.sync_copy(pool_v,
                        out.at[pl.ds(b0, _SAMP),
                               pl.ds((_NTA + w) * _DIM, _DIM)])


_mesh = plsc.VectorSubcoreMesh(core_axis_name="c", subcore_axis_name="s")
_sc_params = pltpu.CompilerParams(use_tc_tiling_on_sc=False)

_sc_pool_b_call = _make_plain_pool(_NTB)
_sc_pool_c_call = _make_plain_pool(_NTC)

_sc_pool_a_call = pl.kernel(
    _sc_pool_a,
    out_type=jax.ShapeDtypeStruct((_B, (_NTA + _NWT) * _DIM), jnp.float32),
    mesh=_mesh,
    compiler_params=_sc_params,
    scratch_types=[
        pltpu.VMEM((_L, _CH), jnp.int32),
        pltpu.VMEM((_SAMP * _L, _DIM), jnp.float32),
        pltpu.VMEM((_SAMP * _L,), jnp.float32),
        pltpu.VMEM((_SAMP, _DIM), jnp.float32),
        pltpu.SemaphoreType.DMA,
    ],
)


_PDA = (_NTA + _NWT) * _DIM
_PDB = _NTB * _DIM
_PDC = _NTC * _DIM
_OB = 8 + _NTA * _DIM
_OC = _OB + _NTB * _DIM
_OW = 8 + _NT * _DIM


def _tc_body(ff, pa, pb, pc, wd, bd, wo, bo, out):
    dense = jnp.dot(ff[...], wd[...], preferred_element_type=jnp.float32) + bd[...]
    r = jnp.dot(dense, wo[0:8, :], preferred_element_type=jnp.float32)
    r = r + jnp.dot(pa[:, :_NTA * _DIM], wo[8:_OB, :],
                    preferred_element_type=jnp.float32)
    r = r + jnp.dot(pa[:, _NTA * _DIM:], wo[_OW:, :],
                    preferred_element_type=jnp.float32)
    r = r + jnp.dot(pb[...], wo[_OB:_OC, :], preferred_element_type=jnp.float32)
    r = r + jnp.dot(pc[...], wo[_OC:_OW, :], preferred_element_type=jnp.float32)
    out[...] = r + bo[...]


_BLK = 512


def _tc_mix(ff, pa, pb, pc, wd, bd, wo, bo):
    return pl.pallas_call(
        _tc_body,
        grid=(_B // _BLK,),
        in_specs=[
            pl.BlockSpec((_BLK, _NF), lambda i: (i, 0)),
            pl.BlockSpec((_BLK, _PDA), lambda i: (i, 0)),
            pl.BlockSpec((_BLK, _PDB), lambda i: (i, 0)),
            pl.BlockSpec((_BLK, _PDC), lambda i: (i, 0)),
            pl.BlockSpec((_NF, 8), lambda i: (0, 0)),
            pl.BlockSpec((1, 8), lambda i: (0, 0)),
            pl.BlockSpec((8 + _PD, _DIM), lambda i: (0, 0)),
            pl.BlockSpec((1, _DIM), lambda i: (0, 0)),
        ],
        out_specs=pl.BlockSpec((_BLK, _DIM), lambda i: (i, 0)),
        out_shape=jax.ShapeDtypeStruct((_B, _DIM), jnp.float32),
    )(ff, pa, pb, pc, wd, bd, wo, bo)


def _remap(idxT, nt):
    tb = (jnp.arange(nt, dtype=jnp.int32) * (_VS // _VC))[:, None, None]
    vb = idxT // _VC
    vr = idxT - vb * _VC
    return ((tb + vb) * _RPB + vr % _RPB) * 8 + vr // _RPB


def kernel(float_features, idlist_indices, idscore_indices, idscore_weights,
           emb_tables, w_emb_tables, W_dense, b_dense, W_over, b_over):
    idlT = jnp.transpose(idlist_indices, (0, 2, 1))
    idlT_a = _remap(idlT[:_NTA], _NTA)
    idlT_b = _remap(idlT[_NTA:_NTA + _NTB], _NTB)
    idlT_c = _remap(idlT[_NTA + _NTB:], _NTC)
    idsT = _remap(jnp.transpose(idscore_indices, (0, 2, 1)), _NWT)
    wts = idscore_weights.reshape(_NWT, _B * _L)
    embT = jnp.transpose(emb_tables, (0, 2, 1))
    wemb = _fmt(jnp.transpose(w_emb_tables, (0, 2, 1)), _NWT
                ).reshape(_NWT * _VS, _DIM)
    emb_a = _fmt(embT, _NTA, 0).reshape(_NTA * _VS, _DIM)
    emb_b = _fmt(embT, _NTB, _NTA).reshape(_NTB * _VS, _DIM)
    emb_c = _fmt(embT, _NTC, _NTA + _NTB).reshape(_NTC * _VS, _DIM)
    pa = _sc_pool_a_call(idlT_a, idsT, wts, emb_a, wemb)
    pb = _sc_pool_b_call(idlT_b, emb_b)
    pc = _sc_pool_c_call(idlT_c, emb_c)
    return _tc_mix(float_features, pa, pb, pc, W_dense, b_dense.reshape(1, 8),
                   W_over, b_over.reshape(1, 16))
